# baseline (speedup 1.0000x reference)
import jax
import jax.numpy as jnp
from jax import lax
from jax.experimental import pallas as pl
from jax.experimental.pallas import tpu as pltpu
from jax.experimental.pallas import tpu_sc as plsc

_N, _C = 65536, 256
_NW = 32
_ROWS_PER_W = _N // _NW
_R = 128
_NCH = _ROWS_PER_W // _R


def _sc_body(x_hbm, out_hbm, buf, sem_ld, sem_st):
    wid = lax.axis_index("s") * 2 + lax.axis_index("c")
    base = wid * _ROWS_PER_W

    def chunk(k, carry):
        row = base + k * _R
        pltpu.async_copy(x_hbm.at[pl.ds(row, _R)], buf, sem_ld).wait()

        @pl.when(jnp.logical_and(wid == 0, k == 0))
        def _patch():
            v = buf[0, pl.ds(0, 16)]
            lane = lax.iota(jnp.int32, 16)
            buf[0, pl.ds(0, 16)] = jnp.where(lane == 0,
                                             jnp.float32(100.0), v)

        c1 = pltpu.async_copy(buf, out_hbm.at[pl.ds(row, _R)], sem_st)
        c2 = pltpu.async_copy(buf, out_hbm.at[pl.ds(_N + row, _R)], sem_st)
        c1.wait()
        c2.wait()
        return carry

    lax.fori_loop(0, _NCH, chunk, 0)


def kernel(x):
    mesh = plsc.VectorSubcoreMesh(core_axis_name="c", subcore_axis_name="s")
    f = pl.kernel(
        _sc_body,
        out_type=jax.ShapeDtypeStruct((2 * _N, _C), jnp.float32),
        mesh=mesh,
        scratch_types=[
            pltpu.VMEM((_R, _C), jnp.float32),
            pltpu.SemaphoreType.DMA,
            pltpu.SemaphoreType.DMA,
        ],
    )
    return f(x)
